# Initial kernel scaffold; baseline (speedup 1.0000x reference)
import jax
import jax.numpy as jnp
from jax.experimental import pallas as pl


def kernel(x, seed_inverse_ids):
    raise NotImplementedError("write your pallas kernel here")


# speedup vs baseline: 2.9027x; 2.9027x over previous
import jax
import jax.numpy as jnp
from jax import lax
from jax.experimental import pallas as pl
from jax.experimental.pallas import tpu as pltpu
from jax.experimental.pallas import tpu_sc as plsc

_NC = 2
_NS = 16
_NW = _NC * _NS
_C = 80


def _gather_body(x_hbm, idx_hbm, out_hbm, idx_v, rows_v, sem):
    bpw = idx_hbm.shape[0] // _NW
    num_chunks = bpw // _C
    wid = lax.axis_index("s") * _NC + lax.axis_index("c")
    base0 = wid * bpw

    def body(j, carry):
        base = base0 + j * _C
        pltpu.sync_copy(idx_hbm.at[pl.ds(base, _C)], idx_v)
        pltpu.async_copy(x_hbm.at[idx_v], rows_v, sem).wait()
        pltpu.sync_copy(rows_v, out_hbm.at[pl.ds(base, _C)])
        return carry

    lax.fori_loop(0, num_chunks, body, 0)


def kernel(x, seed_inverse_ids):
    idx = seed_inverse_ids.astype(jnp.int32)
    B = idx.shape[0]
    mesh = plsc.VectorSubcoreMesh(core_axis_name="c", subcore_axis_name="s")
    k = pl.kernel(
        _gather_body,
        mesh=mesh,
        out_type=jax.ShapeDtypeStruct((B, x.shape[1]), x.dtype),
        scratch_types=[
            pltpu.VMEM((_C,), jnp.int32),
            pltpu.VMEM((_C, x.shape[1]), jnp.float32),
            pltpu.SemaphoreType.DMA,
        ],
    )
    return k(x, idx)


# speedup vs baseline: 6.0066x; 2.0693x over previous
import jax
import jax.numpy as jnp
from jax import lax
from jax.experimental import pallas as pl
from jax.experimental.pallas import tpu as pltpu
from jax.experimental.pallas import tpu_sc as plsc

_NC = 2
_NS = 16
_NW = _NC * _NS
_C = 80
_NBUF = 5


def _gather_body(x_hbm, idx_hbm, out_hbm, idx_v, rows_v, gsem, ssem):
    bpw = idx_hbm.shape[0] // _NW
    nchunks = bpw // _C
    ngroups = nchunks // _NBUF
    wid = lax.axis_index("s") * _NC + lax.axis_index("c")
    base0 = wid * bpw

    def idx_load(j, b):
        pltpu.sync_copy(idx_hbm.at[pl.ds(base0 + j * _C, _C)], idx_v.at[b])

    def gather(b):
        return pltpu.make_async_copy(
            x_hbm.at[idx_v.at[b]], rows_v.at[b], gsem.at[b])

    def store(j, b):
        return pltpu.make_async_copy(
            rows_v.at[b], out_hbm.at[pl.ds(base0 + j * _C, _C)], ssem.at[b])

    for b in range(_NBUF):
        idx_load(b, b)
        gather(b).start()

    def body(g, carry):
        for b in range(_NBUF):
            j = g * _NBUF + b
            gather(b).wait()
            store(j, b).start()
            idx_load(j + _NBUF, b)
            store(j, b).wait()
            gather(b).start()
        return carry

    lax.fori_loop(0, ngroups - 1, body, 0)

    tail = (ngroups - 1) * _NBUF
    for b in range(_NBUF):
        gather(b).wait()
        store(tail + b, b).start()
    for b in range(_NBUF):
        store(tail + b, b).wait()


def kernel(x, seed_inverse_ids):
    idx = seed_inverse_ids.astype(jnp.int32)
    B = idx.shape[0]
    mesh = plsc.VectorSubcoreMesh(core_axis_name="c", subcore_axis_name="s")
    k = pl.kernel(
        _gather_body,
        mesh=mesh,
        out_type=jax.ShapeDtypeStruct((B, x.shape[1]), x.dtype),
        scratch_types=[
            pltpu.VMEM((_NBUF, _C), jnp.int32),
            pltpu.VMEM((_NBUF, _C, x.shape[1]), jnp.float32),
            pltpu.SemaphoreType.DMA((_NBUF,)),
            pltpu.SemaphoreType.DMA((_NBUF,)),
        ],
    )
    return k(x, idx)


# speedup vs baseline: 6.1137x; 1.0178x over previous
import jax
import jax.numpy as jnp
from jax import lax
from jax.experimental import pallas as pl
from jax.experimental.pallas import tpu as pltpu
from jax.experimental.pallas import tpu_sc as plsc

_NC = 2
_NS = 16
_NW = _NC * _NS
_C = 80
_NBUF = 5


def _gather_body(x_hbm, idx_hbm, out_hbm, idx_v, rows_v, gsem, ssem):
    bpw = idx_hbm.shape[0] // _NW
    nchunks = bpw // _C
    ngroups = nchunks // _NBUF
    wid = lax.axis_index("s") * _NC + lax.axis_index("c")
    base0 = wid * bpw

    pltpu.sync_copy(idx_hbm.at[pl.ds(base0, bpw)], idx_v)

    def gather(j, b):
        return pltpu.make_async_copy(
            x_hbm.at[idx_v.at[pl.ds(j * _C, _C)]], rows_v.at[b], gsem.at[b])

    def store(j, b):
        return pltpu.make_async_copy(
            rows_v.at[b], out_hbm.at[pl.ds(base0 + j * _C, _C)], ssem.at[b])

    for b in range(_NBUF):
        gather(b, b).start()

    def body(g, carry):
        for b in range(_NBUF):
            j = g * _NBUF + b
            gather(j, b).wait()
            store(j, b).start()
            store(j, b).wait()
            gather(j + _NBUF, b).start()
        return carry

    lax.fori_loop(0, ngroups - 1, body, 0)

    tail = (ngroups - 1) * _NBUF
    for b in range(_NBUF):
        gather(tail + b, b).wait()
        store(tail + b, b).start()
    for b in range(_NBUF):
        store(tail + b, b).wait()


def kernel(x, seed_inverse_ids):
    idx = seed_inverse_ids.astype(jnp.int32)
    B = idx.shape[0]
    mesh = plsc.VectorSubcoreMesh(core_axis_name="c", subcore_axis_name="s")
    k = pl.kernel(
        _gather_body,
        mesh=mesh,
        out_type=jax.ShapeDtypeStruct((B, x.shape[1]), x.dtype),
        scratch_types=[
            pltpu.VMEM((B // _NW,), jnp.int32),
            pltpu.VMEM((_NBUF, _C, x.shape[1]), jnp.float32),
            pltpu.SemaphoreType.DMA((_NBUF,)),
            pltpu.SemaphoreType.DMA((_NBUF,)),
        ],
    )
    return k(x, idx)
